# baseline (speedup 1.0000x reference)
import functools

import jax
import jax.numpy as jnp
from jax import lax
from jax.experimental import pallas as pl
from jax.experimental.pallas import tpu as pltpu
from jax.experimental.pallas import tpu_sc as plsc

MAXP = 32
NIDX = 2 * MAXP + 1
ZD = 128
LS = 512
NW = 16
RPW = LS // NW
WIN = LS + RPW

_mesh = plsc.VectorSubcoreMesh(core_axis_name="c", subcore_axis_name="s")


@functools.partial(
    pl.kernel,
    mesh=_mesh,
    out_type=jax.ShapeDtypeStruct((LS, LS, ZD), jnp.float32),
    scratch_types=[
        pltpu.VMEM((NIDX, ZD), jnp.float32),
        pltpu.VMEM((WIN, ZD), jnp.float32),
        pltpu.SemaphoreType.DMA,
    ],
)
def _rel_pos_bias(table_hbm, out_hbm, table_v, win_v, sem):
    cid = lax.axis_index("c")

    @pl.when(cid == 0)
    def _():
        wid = lax.axis_index("s")
        row0 = wid * RPW
        m0 = (LS - 1) - (row0 + (RPW - 1))

        pltpu.sync_copy(table_hbm, table_v)

        @plsc.parallel_loop(0, WIN, 1, unroll=8)
        def _build(r):
            c = jnp.clip((LS - 1) - (m0 + r), -MAXP, MAXP) + MAXP
            for k in range(ZD // 16):
                win_v[r, pl.ds(k * 16, 16)] = table_v[c, pl.ds(k * 16, 16)]

        copies = []
        for t in range(RPW):
            cp = pltpu.make_async_copy(
                win_v.at[pl.ds((RPW - 1) - t, LS)], out_hbm.at[row0 + t], sem
            )
            cp.start()
            copies.append(cp)
        for cp in copies:
            cp.wait()


def kernel(L, embed_table):
    return _rel_pos_bias(embed_table)


# speedup vs baseline: 2.0449x; 2.0449x over previous
import functools

import jax
import jax.numpy as jnp
from jax import lax
from jax.experimental import pallas as pl
from jax.experimental.pallas import tpu as pltpu

MAXP = 32
NIDX = 2 * MAXP + 1
ZD = 128
LS = 512
BR = 8
NG = LS // BR
NR2 = 2 * LS - 1


def _tc_body(table_ref, out_ref, r2_ref):
    g = pl.program_id(0)

    @pl.when(g == 0)
    def _build():
        r2_ref[pl.ds(0, LS - MAXP - 1), :] = jnp.broadcast_to(
            table_ref[pl.ds(NIDX - 1, 1), :], (LS - MAXP - 1, ZD)
        )

        def mid(r, carry):
            r2_ref[pl.ds(LS - MAXP - 1 + r, 1), :] = table_ref[pl.ds(NIDX - 1 - r, 1), :]
            return carry

        lax.fori_loop(0, NIDX, mid, 0)
        r2_ref[pl.ds(LS + MAXP, LS - MAXP - 1), :] = jnp.broadcast_to(
            table_ref[pl.ds(0, 1), :], (LS - MAXP - 1, ZD)
        )

    for t in range(BR):
        i = g * BR + t
        out_ref[t] = r2_ref[pl.ds((LS - 1) - i, LS), :]


@functools.partial(jax.jit, static_argnums=0)
def _run(_unused, table):
    return pl.pallas_call(
        _tc_body,
        grid=(NG,),
        in_specs=[pl.BlockSpec((NIDX, ZD), lambda g: (0, 0))],
        out_specs=pl.BlockSpec((BR, LS, ZD), lambda g: (g, 0, 0)),
        out_shape=jax.ShapeDtypeStruct((LS, LS, ZD), jnp.float32),
        scratch_shapes=[pltpu.VMEM((NR2 + 1, ZD), jnp.float32)],
        compiler_params=pltpu.CompilerParams(
            dimension_semantics=("arbitrary",),
        ),
    )(table)


def kernel(L, embed_table):
    return _run(0, embed_table)
